# baseline (speedup 1.0000x reference)
import functools

import jax
import jax.numpy as jnp
from jax import lax
from jax.experimental import pallas as pl
from jax.experimental.pallas import tpu as pltpu
from jax.experimental.pallas import tpu_sc as plsc

NUM_ROWS = 128
ROW_LEN = 100000
LANES = 16
NUM_CORES = 2
NUM_SUBCORES = 16
NUM_WORKERS = NUM_CORES * NUM_SUBCORES
GROUPS = NUM_ROWS // LANES
STRIPE = 3128
CHUNK_V = 136
CHUNKS = STRIPE // CHUNK_V
LAST_STRIPE_BASE = ROW_LEN - STRIPE

_NEG_INF = float("-inf")


def _consume_chunk(buf_ref, idx_base, carry):
    def body(j, c):
        ms, its = c
        ms, its = list(ms), list(its)
        it_vec = jnp.broadcast_to(idx_base + j, (LANES,)).astype(jnp.int32)
        for g in range(GROUPS):
            v = buf_ref[j, pl.ds(g * LANES, LANES)]
            cmp = v > ms[g]
            ms[g] = jnp.maximum(ms[g], v)
            its[g] = jnp.where(cmp, it_vec, its[g])
        return tuple(ms), tuple(its)

    return lax.fori_loop(0, CHUNK_V, body, carry)


def _build_sc_call():
    mesh = plsc.VectorSubcoreMesh(core_axis_name="c", subcore_axis_name="s",
                                  num_cores=NUM_CORES,
                                  num_subcores=NUM_SUBCORES)

    @functools.partial(
        pl.kernel,
        out_type=(
            jax.ShapeDtypeStruct((NUM_CORES * GROUPS, LANES), jnp.float32),
            jax.ShapeDtypeStruct((NUM_CORES * GROUPS, LANES), jnp.int32),
            jax.ShapeDtypeStruct((NUM_WORKERS, GROUPS, LANES), jnp.float32),
            jax.ShapeDtypeStruct((NUM_WORKERS, GROUPS, LANES), jnp.int32),
        ),
        mesh=mesh,
        scratch_types=[
            pltpu.VMEM((CHUNK_V, NUM_ROWS), jnp.float32),
            pltpu.VMEM((CHUNK_V, NUM_ROWS), jnp.float32),
            pltpu.VMEM((CHUNK_V, NUM_ROWS), jnp.float32),
            pltpu.VMEM((CHUNK_V, NUM_ROWS), jnp.float32),
            pltpu.VMEM((CHUNK_V, NUM_ROWS), jnp.float32),
            pltpu.VMEM((GROUPS, LANES), jnp.float32),
            pltpu.VMEM((GROUPS, LANES), jnp.int32),
            pltpu.VMEM((NUM_SUBCORES, GROUPS, LANES), jnp.float32),
            pltpu.VMEM((NUM_SUBCORES, GROUPS, LANES), jnp.int32),
            pltpu.VMEM((LANES,), jnp.int32),
            pltpu.VMEM((LANES,), jnp.int32),
            pltpu.VMEM((LANES,), jnp.float32),
            pltpu.SemaphoreType.DMA,
            pltpu.SemaphoreType.DMA,
            pltpu.SemaphoreType.DMA,
            pltpu.SemaphoreType.DMA,
            pltpu.SemaphoreType.DMA,
            pltpu.SemaphoreType.DMA,
        ],
    )
    def sc_kernel(yt_hbm, table_hbm, out_val_hbm, out_word_hbm,
                  stage_val_hbm, stage_idx_hbm,
                  buf0, buf1, buf2, buf3, buf4, cand_val, cand_idx,
                  merge_val, merge_idx,
                  idx_buf, word_buf, val_buf,
                  sem0, sem1, sem2, sem3, sem4, gsem):
        core = lax.axis_index("c")
        sub = lax.axis_index("s")
        stripe_rank = core * NUM_SUBCORES + sub
        sb = jnp.minimum(stripe_rank * STRIPE, LAST_STRIPE_BASE)
        sb = pl.multiple_of(sb, 8)
        bufs = (buf0, buf1, buf2, buf3, buf4)
        sems = (sem0, sem1, sem2, sem3, sem4)
        NBUF = len(bufs)

        def start_copy(c):
            return pltpu.async_copy(
                yt_hbm.at[pl.ds(sb + c * CHUNK_V, CHUNK_V)],
                bufs[c % NBUF], sems[c % NBUF])

        ms = tuple(jnp.full((LANES,), _NEG_INF, jnp.float32)
                   for _ in range(GROUPS))
        its = tuple(jnp.zeros((LANES,), jnp.int32) for _ in range(GROUPS))
        carry = (ms, its)
        handles = {c: start_copy(c) for c in range(NBUF - 1)}
        for c in range(CHUNKS):
            handles.pop(c).wait()
            if c + NBUF - 1 < CHUNKS:
                handles[c + NBUF - 1] = start_copy(c + NBUF - 1)
            carry = _consume_chunk(bufs[c % NBUF], sb + c * CHUNK_V, carry)
        ms, its = carry

        for g in range(GROUPS):
            cand_val[g, ...] = ms[g]
            cand_idx[g, ...] = its[g]
        pltpu.sync_copy(cand_val, stage_val_hbm.at[stripe_rank])
        pltpu.sync_copy(cand_idx, stage_idx_hbm.at[stripe_rank])
        plsc.subcore_barrier()

        @pl.when(sub < GROUPS)
        def _merge():
            g = sub
            pltpu.sync_copy(
                stage_val_hbm.at[pl.ds(core * NUM_SUBCORES, NUM_SUBCORES)],
                merge_val)
            pltpu.sync_copy(
                stage_idx_hbm.at[pl.ds(core * NUM_SUBCORES, NUM_SUBCORES)],
                merge_idx)
            bv = merge_val[0, g, ...]
            bi = merge_idx[0, g, ...]
            for t in range(1, NUM_SUBCORES):
                v = merge_val[t, g, ...]
                i = merge_idx[t, g, ...]
                take = jnp.logical_or(
                    v > bv, jnp.logical_and(v == bv, i < bi))
                bv = jnp.where(take, v, bv)
                bi = jnp.where(take, i, bi)
            idx_buf[...] = bi
            val_buf[...] = bv
            pltpu.async_copy(table_hbm.at[idx_buf], word_buf, gsem).wait()
            out_row = core * GROUPS + g
            pltpu.sync_copy(val_buf, out_val_hbm.at[out_row])
            pltpu.sync_copy(word_buf, out_word_hbm.at[out_row])

    return sc_kernel


_sc_call = _build_sc_call()


@jax.jit
def kernel(y_pred, word_table):
    vals, words, _, _ = _sc_call(y_pred.T, word_table)
    v0 = vals[:GROUPS].reshape(-1)
    v1 = vals[GROUPS:].reshape(-1)
    w0 = words[:GROUPS].reshape(-1)
    w1 = words[GROUPS:].reshape(-1)
    return jnp.where(v0 >= v1, w0, w1)


# speedup vs baseline: 1.0107x; 1.0107x over previous
import functools

import jax
import jax.numpy as jnp
from jax import lax
from jax.experimental import pallas as pl
from jax.experimental.pallas import tpu as pltpu
from jax.experimental.pallas import tpu_sc as plsc

NUM_ROWS = 128
ROW_LEN = 100000
LANES = 16
NUM_CORES = 2
NUM_SUBCORES = 16
NUM_WORKERS = NUM_CORES * NUM_SUBCORES
GROUPS = NUM_ROWS // LANES
STRIPE = 3128
CHUNK_V = 184
CHUNKS = STRIPE // CHUNK_V
LAST_STRIPE_BASE = ROW_LEN - STRIPE

_NEG_INF = float("-inf")


def _consume_chunk(buf_ref, idx_base, carry):
    def body(j, c):
        ms, its = c
        ms, its = list(ms), list(its)
        it_vec = jnp.broadcast_to(idx_base + j, (LANES,)).astype(jnp.int32)
        for g in range(GROUPS):
            v = buf_ref[j, pl.ds(g * LANES, LANES)]
            cmp = v > ms[g]
            ms[g] = jnp.maximum(ms[g], v)
            its[g] = jnp.where(cmp, it_vec, its[g])
        return tuple(ms), tuple(its)

    return lax.fori_loop(0, CHUNK_V, body, carry)


def _build_sc_call():
    mesh = plsc.VectorSubcoreMesh(core_axis_name="c", subcore_axis_name="s",
                                  num_cores=NUM_CORES,
                                  num_subcores=NUM_SUBCORES)

    @functools.partial(
        pl.kernel,
        out_type=(
            jax.ShapeDtypeStruct((NUM_CORES * GROUPS, LANES), jnp.float32),
            jax.ShapeDtypeStruct((NUM_CORES * GROUPS, LANES), jnp.int32),
            jax.ShapeDtypeStruct((NUM_WORKERS, GROUPS, LANES), jnp.float32),
            jax.ShapeDtypeStruct((NUM_WORKERS, GROUPS, LANES), jnp.int32),
        ),
        mesh=mesh,
        scratch_types=[
            pltpu.VMEM((CHUNK_V, NUM_ROWS), jnp.float32),
            pltpu.VMEM((CHUNK_V, NUM_ROWS), jnp.float32),
            pltpu.VMEM((CHUNK_V, NUM_ROWS), jnp.float32),
            pltpu.VMEM((CHUNK_V, NUM_ROWS), jnp.float32),
            pltpu.VMEM((GROUPS, LANES), jnp.float32),
            pltpu.VMEM((GROUPS, LANES), jnp.int32),
            pltpu.VMEM((NUM_SUBCORES, GROUPS, LANES), jnp.float32),
            pltpu.VMEM((NUM_SUBCORES, GROUPS, LANES), jnp.int32),
            pltpu.VMEM((LANES,), jnp.int32),
            pltpu.VMEM((LANES,), jnp.int32),
            pltpu.VMEM((LANES,), jnp.float32),
            pltpu.SemaphoreType.DMA,
            pltpu.SemaphoreType.DMA,
            pltpu.SemaphoreType.DMA,
            pltpu.SemaphoreType.DMA,
            pltpu.SemaphoreType.DMA,
        ],
    )
    def sc_kernel(yt_hbm, table_hbm, out_val_hbm, out_word_hbm,
                  stage_val_hbm, stage_idx_hbm,
                  buf0, buf1, buf2, buf3, cand_val, cand_idx,
                  merge_val, merge_idx,
                  idx_buf, word_buf, val_buf,
                  sem0, sem1, sem2, sem3, gsem):
        core = lax.axis_index("c")
        sub = lax.axis_index("s")
        stripe_rank = core * NUM_SUBCORES + sub
        sb = jnp.minimum(stripe_rank * STRIPE, LAST_STRIPE_BASE)
        sb = pl.multiple_of(sb, 8)
        bufs = (buf0, buf1, buf2, buf3)
        sems = (sem0, sem1, sem2, sem3)
        NBUF = len(bufs)

        def start_copy(c):
            return pltpu.async_copy(
                yt_hbm.at[pl.ds(sb + c * CHUNK_V, CHUNK_V)],
                bufs[c % NBUF], sems[c % NBUF])

        ms = tuple(jnp.full((LANES,), _NEG_INF, jnp.float32)
                   for _ in range(GROUPS))
        its = tuple(jnp.zeros((LANES,), jnp.int32) for _ in range(GROUPS))
        carry = (ms, its)
        handles = {c: start_copy(c) for c in range(NBUF - 1)}
        for c in range(CHUNKS):
            handles.pop(c).wait()
            if c + NBUF - 1 < CHUNKS:
                handles[c + NBUF - 1] = start_copy(c + NBUF - 1)
            carry = _consume_chunk(bufs[c % NBUF], sb + c * CHUNK_V, carry)
        ms, its = carry

        for g in range(GROUPS):
            cand_val[g, ...] = ms[g]
            cand_idx[g, ...] = its[g]
        pltpu.sync_copy(cand_val, stage_val_hbm.at[stripe_rank])
        pltpu.sync_copy(cand_idx, stage_idx_hbm.at[stripe_rank])
        plsc.subcore_barrier()

        @pl.when(sub < GROUPS)
        def _merge():
            g = sub
            pltpu.sync_copy(
                stage_val_hbm.at[pl.ds(core * NUM_SUBCORES, NUM_SUBCORES)],
                merge_val)
            pltpu.sync_copy(
                stage_idx_hbm.at[pl.ds(core * NUM_SUBCORES, NUM_SUBCORES)],
                merge_idx)
            bv = merge_val[0, g, ...]
            bi = merge_idx[0, g, ...]
            for t in range(1, NUM_SUBCORES):
                v = merge_val[t, g, ...]
                i = merge_idx[t, g, ...]
                take = jnp.logical_or(
                    v > bv, jnp.logical_and(v == bv, i < bi))
                bv = jnp.where(take, v, bv)
                bi = jnp.where(take, i, bi)
            idx_buf[...] = bi
            val_buf[...] = bv
            pltpu.async_copy(table_hbm.at[idx_buf], word_buf, gsem).wait()
            out_row = core * GROUPS + g
            pltpu.sync_copy(val_buf, out_val_hbm.at[out_row])
            pltpu.sync_copy(word_buf, out_word_hbm.at[out_row])

    return sc_kernel


_sc_call = _build_sc_call()


@jax.jit
def kernel(y_pred, word_table):
    vals, words, _, _ = _sc_call(y_pred.T, word_table)
    v0 = vals[:GROUPS].reshape(-1)
    v1 = vals[GROUPS:].reshape(-1)
    w0 = words[:GROUPS].reshape(-1)
    w1 = words[GROUPS:].reshape(-1)
    return jnp.where(v0 >= v1, w0, w1)
